# Initial kernel scaffold; baseline (speedup 1.0000x reference)
import jax
import jax.numpy as jnp
from jax.experimental import pallas as pl


def kernel(x, edge_attr, edge_index, batch, params, Wlin, blin, Wout, bout):
    raise NotImplementedError("write your pallas kernel here")


# speedup vs baseline: 23.9911x; 23.9911x over previous
import functools

import jax
import jax.numpy as jnp
from jax import lax
from jax.experimental import pallas as pl
from jax.experimental.pallas import tpu as pltpu
from jax.experimental.pallas import tpu_sc as plsc

N = 10000
E = 320000
EDIM = 16
H = 4
C = 32
HC = H * C
NG = 64
NL = 2
DN = 128

INV_SQRT_C = 1.0 / (32.0 ** 0.5)

NUM_CORES = 2
NUM_SUBCORES = 16
NW = NUM_CORES * NUM_SUBCORES
EPW = E // NW
CH = 40
NCHUNK = EPW // CH
NP = 10240
ND = NP // 8
NROWS = NP + ND
RPT = NROWS // NUM_SUBCORES

BN = 1000
BE = 4000


def _proj_body(h_ref, wq_ref, wkv_ref, ws_ref, bq_ref, bkv_ref, bs_ref,
               q_ref, kv_ref, s_ref):
    h = h_ref[...]
    q = jnp.dot(h, wq_ref[...], preferred_element_type=jnp.float32)
    q_ref[...] = (q + bq_ref[0:1, :]) * INV_SQRT_C
    kv = jnp.dot(h, wkv_ref[...], preferred_element_type=jnp.float32)
    kv_ref[...] = kv + bkv_ref[0:1, :]
    s = jnp.dot(h, ws_ref[...], preferred_element_type=jnp.float32)
    s_ref[...] = s + bs_ref[0:1, :]


def _proj(h, wq, wkv, ws, bq, bkv, bs):
    grid = (N // BN,)
    return pl.pallas_call(
        _proj_body,
        grid=grid,
        in_specs=[
            pl.BlockSpec((BN, HC), lambda i: (i, 0)),
            pl.BlockSpec((HC, HC), lambda i: (0, 0)),
            pl.BlockSpec((HC, 2 * HC), lambda i: (0, 0)),
            pl.BlockSpec((HC, HC), lambda i: (0, 0)),
            pl.BlockSpec((8, HC), lambda i: (0, 0)),
            pl.BlockSpec((8, 2 * HC), lambda i: (0, 0)),
            pl.BlockSpec((8, HC), lambda i: (0, 0)),
        ],
        out_specs=[
            pl.BlockSpec((BN, HC), lambda i: (i, 0)),
            pl.BlockSpec((BN, 2 * HC), lambda i: (i, 0)),
            pl.BlockSpec((BN, HC), lambda i: (i, 0)),
        ],
        out_shape=[
            jax.ShapeDtypeStruct((N, HC), jnp.float32),
            jax.ShapeDtypeStruct((N, 2 * HC), jnp.float32),
            jax.ShapeDtypeStruct((N, HC), jnp.float32),
        ],
    )(h, wq, wkv, ws, bq, bkv, bs)


def _e_body(ea_ref, we_ref, e_ref):
    e_ref[...] = jnp.dot(ea_ref[...], we_ref[...],
                         preferred_element_type=jnp.float32)


def _e_proj(edge_attr, we):
    return pl.pallas_call(
        _e_body,
        grid=(E // BE,),
        in_specs=[
            pl.BlockSpec((BE, EDIM), lambda i: (i, 0)),
            pl.BlockSpec((EDIM, HC), lambda i: (0, 0)),
        ],
        out_specs=pl.BlockSpec((BE, HC), lambda i: (i, 0)),
        out_shape=jax.ShapeDtypeStruct((E, HC), jnp.float32),
    )(edge_attr, we)


def _lane_perm(x, idx):
    dn = lax.GatherDimensionNumbers(offset_dims=(), collapsed_slice_dims=(0,),
                                    start_index_map=(0,))
    return lax.gather(x, idx[:, None], dn, slice_sizes=(1,),
                      mode=lax.GatherScatterMode.PROMISE_IN_BOUNDS)

def _edge_sc_body(q_hbm, kv_hbm, e_hbm, src_hbm, dst_hbm, zn_hbm,
                  num_hbm,
                  srcv, dstv, slotv, dstw, qb, kvb, eb, msgb, denw, accn, sem):
    cid = lax.axis_index("c")
    sid = lax.axis_index("s")
    wid = cid * NUM_SUBCORES + sid
    ebase = wid * EPW
    rbase = sid * RPT

    pltpu.sync_copy(zn_hbm.at[pl.ds(rbase, RPT)], accn.at[pl.ds(rbase, RPT)])
    plsc.subcore_barrier()

    def chunk_body(ci, carry):
        base = ebase + ci * CH
        pltpu.sync_copy(src_hbm.at[pl.ds(base, CH)], srcv)
        pltpu.sync_copy(dst_hbm.at[pl.ds(base, CH)], dstv)
        cq = pltpu.async_copy(q_hbm.at[dstv], qb, sem)
        ckv = pltpu.async_copy(kv_hbm.at[srcv], kvb, sem)
        ce = pltpu.async_copy(e_hbm.at[pl.ds(base, CH)], eb, sem)
        cq.wait()
        ckv.wait()
        ce.wait()

        for off in (0, 16, 24):
            dv = dstv[pl.ds(off, 16)]
            dstw[pl.ds(off, 16)] = NP + lax.shift_right_logical(dv, 3)
            slotv[pl.ds(off, 16)] = jnp.bitwise_and(dv, 7)

        U = 4

        def edge_body(j0, carry2):
            lane = lax.iota(jnp.int32, 16)
            for u in range(U):
                j = j0 * U + u
                qc = [qb[j, pl.ds(16 * c, 16)] for c in range(8)]
                ec = [eb[j, pl.ds(16 * c, 16)] for c in range(8)]
                kc = [kvb[j, pl.ds(16 * c, 16)] for c in range(8)]
                vc = [kvb[j, pl.ds(HC + 16 * c, 16)] for c in range(8)]
                t = [qc[c] * (kc[c] + ec[c]) for c in range(8)]
                wv = []
                for hh in range(4):
                    s = t[2 * hh] + t[2 * hh + 1]
                    for sh in (8, 4, 2, 1):
                        s = s + _lane_perm(s, lane ^ sh)
                    wv.append(jnp.exp(s))
                for c in range(8):
                    msgb[j, pl.ds(16 * c, 16)] = (vc[c] + ec[c]) * wv[c // 2]
                d = jnp.where(lane == 0, wv[0],
                    jnp.where(lane == 1, wv[1],
                    jnp.where(lane == 2, wv[2],
                    jnp.where(lane == 3, wv[3],
                              jnp.zeros((16,), jnp.float32)))))
                sw = slotv[pl.ds(j, 16)]
                slotb = _lane_perm(sw, jnp.zeros((16,), jnp.int32))
                for c in range(8):
                    eq = jnp.maximum(1 - jnp.abs(slotb - c), 0)
                    denw[j, pl.ds(16 * c, 16)] = d * eq.astype(jnp.float32)
            return carry2

        lax.fori_loop(0, CH // U, edge_body, 0)
        pltpu.sync_copy(msgb, accn.at[dstv], add=True)
        pltpu.sync_copy(denw, accn.at[dstw], add=True)
        return carry

    lax.fori_loop(0, NCHUNK, chunk_body, 0)
    plsc.subcore_barrier()
    pltpu.sync_copy(accn.at[pl.ds(rbase, RPT)],
                    num_hbm.at[cid, pl.ds(rbase, RPT)])


def _edge_sc(q, kv, e, src, dst, zn):
    mesh = plsc.VectorSubcoreMesh(core_axis_name="c", subcore_axis_name="s")
    call = pl.kernel(
        _edge_sc_body,
        out_type=[
            jax.ShapeDtypeStruct((NUM_CORES, NROWS, HC), jnp.float32),
        ],
        mesh=mesh,
        scratch_types=[
            pltpu.VMEM((CH,), jnp.int32),
            pltpu.VMEM((CH,), jnp.int32),
            pltpu.VMEM((CH + 16,), jnp.int32),
            pltpu.VMEM((CH,), jnp.int32),
            pltpu.VMEM((CH, HC), jnp.float32),
            pltpu.VMEM((CH, 2 * HC), jnp.float32),
            pltpu.VMEM((CH, HC), jnp.float32),
            pltpu.VMEM((CH, HC), jnp.float32),
            pltpu.VMEM((CH, HC), jnp.float32),
            pltpu.VMEM_SHARED((NROWS, HC), jnp.float32),
            pltpu.SemaphoreType.DMA,
        ],
    )
    return call(q, kv, e, src, dst, zn)


def _epi_body(num_ref, den_ref, sk_ref, wb_ref, h_ref):
    num = num_ref[0] + num_ref[1]
    den = den_ref[0] + den_ref[1]
    rep = jnp.concatenate(
        [jnp.kron(jnp.eye(4, dtype=jnp.float32), jnp.ones((1, C), jnp.float32)),
         jnp.zeros((12, HC), jnp.float32)], axis=0)
    denx = jnp.dot(den, rep, preferred_element_type=jnp.float32)
    out = num / (denx + 1e-16)
    xr = sk_ref[...]
    wa = wb_ref[0:1, :] + wb_ref[2:3, :]
    wx = wb_ref[1:2, :] - wb_ref[2:3, :]
    bpre = jnp.sum(out * wa + xr * wx, axis=1, keepdims=True)
    b = jax.nn.sigmoid(bpre)
    h_ref[...] = jnp.maximum(b * xr + (1.0 - b) * out, 0.0)


def _epilogue(num, den, sk, wb):
    return pl.pallas_call(
        _epi_body,
        grid=(N // BN,),
        in_specs=[
            pl.BlockSpec((NUM_CORES, BN, HC), lambda i: (0, i, 0)),
            pl.BlockSpec((NUM_CORES, BN, 16), lambda i: (0, i, 0)),
            pl.BlockSpec((BN, HC), lambda i: (i, 0)),
            pl.BlockSpec((8, HC), lambda i: (0, 0)),
        ],
        out_specs=pl.BlockSpec((BN, HC), lambda i: (i, 0)),
        out_shape=jax.ShapeDtypeStruct((N, HC), jnp.float32),
    )(num, den, sk, wb)


def _pool_body(h_ref, b_ref, wlin_ref, wout_ref, consts_ref,
               o_ref, mx_ref, sm_ref, ct_ref):
    i = pl.program_id(0)
    nb = pl.num_programs(0)

    @pl.when(i == 0)
    def _init():
        mx_ref[...] = jnp.full((NG, HC), -jnp.inf, jnp.float32)
        sm_ref[...] = jnp.zeros((NG, HC), jnp.float32)
        ct_ref[...] = jnp.zeros((NG, HC), jnp.float32)

    hb = h_ref[...]
    bb = b_ref[...]
    for g in range(NG):
        m = bb == g
        hm = jnp.where(m, hb, -jnp.inf)
        mx_ref[g:g + 1, :] = jnp.maximum(mx_ref[g:g + 1, :],
                                         jnp.max(hm, axis=0, keepdims=True))
        hs = jnp.where(m, hb, 0.0)
        sm_ref[g:g + 1, :] = sm_ref[g:g + 1, :] + jnp.sum(hs, axis=0,
                                                          keepdims=True)
        ct_ref[g:g + 1, :] = ct_ref[g:g + 1, :] + jnp.sum(
            m.astype(jnp.float32))

    @pl.when(i == nb - 1)
    def _final():
        gmax = mx_ref[...]
        gmean = sm_ref[...] / jnp.maximum(ct_ref[...], 1.0)
        z = jnp.concatenate([gmax, gmean], axis=1)
        t = jnp.dot(z, wlin_ref[...], preferred_element_type=jnp.float32)
        t = t + consts_ref[0:1, :]
        o = jnp.dot(t, wout_ref[...], preferred_element_type=jnp.float32)
        o = o + consts_ref[1:2, 0:1]
        o_ref[...] = jax.nn.sigmoid(o)


def _pool_mlp(h, batch2, wlin, wout, consts):
    return pl.pallas_call(
        _pool_body,
        grid=(N // BN,),
        in_specs=[
            pl.BlockSpec((BN, HC), lambda i: (i, 0)),
            pl.BlockSpec((BN, 1), lambda i: (i, 0)),
            pl.BlockSpec((2 * HC, DN), lambda i: (0, 0)),
            pl.BlockSpec((DN, 1), lambda i: (0, 0)),
            pl.BlockSpec((8, HC), lambda i: (0, 0)),
        ],
        out_specs=pl.BlockSpec((NG, 1), lambda i: (0, 0)),
        out_shape=jax.ShapeDtypeStruct((NG, 1), jnp.float32),
        scratch_shapes=[
            pltpu.VMEM((NG, HC), jnp.float32),
            pltpu.VMEM((NG, HC), jnp.float32),
            pltpu.VMEM((NG, HC), jnp.float32),
        ],
    )(h, batch2, wlin, wout, consts)


def _pad8(a):
    return jnp.pad(a, ((0, 8 - a.shape[0]), (0, 0)))


def kernel(x, edge_attr, edge_index, batch, params, Wlin, blin, Wout, bout):
    src = edge_index[0].astype(jnp.int32)
    dst = edge_index[1].astype(jnp.int32)
    batch2 = batch.astype(jnp.int32).reshape(N, 1)
    zn = jnp.zeros((NROWS, HC), jnp.float32)

    h = x
    for l in range(NL + 1):
        p = params[l]
        wkv = jnp.concatenate([p['Wk'], p['Wv']], axis=1)
        bq = _pad8(p['bq'].reshape(1, HC))
        bkv = _pad8(jnp.concatenate([p['bk'], p['bv']]).reshape(1, 2 * HC))
        bs = _pad8(p['bskip'].reshape(1, HC))
        q, kv, sk = _proj(h, p['Wq'], wkv, p['Wskip'], bq, bkv, bs)
        e = _e_proj(edge_attr, p['We'])
        (acc,) = _edge_sc(q, kv, e, src, dst, zn)
        den = acc[:, NP:, :].reshape(NUM_CORES, NP, 16)
        wb = _pad8(p['Wbeta'].reshape(3, HC))
        h = _epilogue(acc, den, sk, wb)

    consts = _pad8(jnp.concatenate(
        [blin.reshape(1, DN),
         jnp.pad(bout.reshape(1, 1), ((0, 0), (0, DN - 1)))], axis=0))
    return _pool_mlp(h, batch2, Wlin, Wout, consts)
